# Initial kernel scaffold; baseline (speedup 1.0000x reference)
import jax
import jax.numpy as jnp
from jax.experimental import pallas as pl


def kernel(relative_position_encoding, features, neighbors):
    raise NotImplementedError("write your pallas kernel here")


# speedup vs baseline: 7.6298x; 7.6298x over previous
import functools

import jax
import jax.numpy as jnp
from jax import lax
from jax.experimental import pallas as pl
from jax.experimental.pallas import tpu as pltpu
from jax.experimental.pallas import tpu_sc as plsc

B, C, N, K = 4, 64, 10000, 16
NK = N * K
NSC = 32
G = 8
Q = 1280
NCHUNK = NK // Q
LANES = 16

_sc_mesh = plsc.VectorSubcoreMesh(core_axis_name="c", subcore_axis_name="s")


@functools.partial(
    pl.kernel,
    mesh=_sc_mesh,
    compiler_params=pltpu.CompilerParams(
        use_tc_tiling_on_sc=False, needs_layout_passes=False
    ),
    out_type=jax.ShapeDtypeStruct((B, 2 * C, NK), jnp.float32),
    scratch_types=(
        [pltpu.VMEM((N,), jnp.float32) for _ in range(G)]
        + [pltpu.VMEM((Q,), jnp.int32), pltpu.VMEM((G, Q), jnp.float32)]
    ),
)
def _sc_gather(feat_hbm, idx_hbm, out_hbm, *refs):
    tabs = refs[0:G]
    idx_v = refs[G]
    out_v = refs[G + 1]
    wid = lax.axis_index("s") * 2 + lax.axis_index("c")
    b = wid // (C // G)
    cg = wid % (C // G)
    for cl in range(G):
        pltpu.sync_copy(
            feat_hbm.at[pl.ds((b * C + cg * G + cl) * N, N)], tabs[cl]
        )

    def chunk(q, carry):
        pltpu.sync_copy(idx_hbm.at[pl.ds(b * NK + q * Q, Q)], idx_v)

        def inner(i, c2):
            iv = idx_v[pl.ds(i * LANES, LANES)]
            for cl in range(G):
                out_v[cl, pl.ds(i * LANES, LANES)] = plsc.load_gather(
                    tabs[cl], [iv]
                )
            return c2

        lax.fori_loop(0, Q // LANES, inner, 0, unroll=2)
        pltpu.sync_copy(
            out_v, out_hbm.at[b, pl.ds(C + cg * G, G), pl.ds(q * Q, Q)]
        )
        return carry

    lax.fori_loop(0, NCHUNK, chunk, 0)


_BJ = 16000


def _copy_body(rpe_ref, out_alias_ref, out_ref):
    del out_alias_ref
    out_ref[...] = rpe_ref[...]


def _tc_fill_rpe(rpe_r, out_r):
    return pl.pallas_call(
        _copy_body,
        grid=(B, NK // _BJ),
        in_specs=[
            pl.BlockSpec((1, C, _BJ), lambda b, j: (b, 0, j)),
            pl.BlockSpec(memory_space=pl.ANY),
        ],
        out_specs=pl.BlockSpec((1, C, _BJ), lambda b, j: (b, 0, j)),
        out_shape=jax.ShapeDtypeStruct((B, 2 * C, NK), jnp.float32),
        input_output_aliases={1: 0},
    )(rpe_r, out_r)


def kernel(relative_position_encoding, features, neighbors):
    feat = features.reshape(B * C * N)
    idx = neighbors.reshape(B * NK)
    rpe_r = relative_position_encoding.reshape(B, C, NK)
    out_r = _sc_gather(feat, idx)
    out_r = _tc_fill_rpe(rpe_r, out_r)
    return out_r.reshape(B, 2 * C, N, K)


# speedup vs baseline: 13.5736x; 1.7790x over previous
import functools

import jax
import jax.numpy as jnp
from jax import lax
from jax.experimental import pallas as pl
from jax.experimental.pallas import tpu as pltpu
from jax.experimental.pallas import tpu_sc as plsc

B, C, N, K = 4, 64, 10000, 16
NK = N * K
NSC = 32
IW = 128
RPC = 4
NROWS = B * NK // IW
NCHUNKS = NROWS // RPC
CHUNK = RPC * IW

_sc_mesh = plsc.VectorSubcoreMesh(core_axis_name="c", subcore_axis_name="s")


@functools.partial(
    pl.kernel,
    mesh=_sc_mesh,
    compiler_params=pltpu.CompilerParams(
        use_tc_tiling_on_sc=False, needs_layout_passes=False
    ),
    out_type=jax.ShapeDtypeStruct((B * NK, C), jnp.float32),
    scratch_types=[
        pltpu.VMEM((RPC, IW), jnp.int32),
        pltpu.VMEM((CHUNK, C), jnp.float32),
        pltpu.SemaphoreType.DMA,
    ],
)
def _sc_gather(ftab_hbm, idx_hbm, gath_hbm, idx_buf, rows_buf, sem):
    wid = lax.axis_index("s") * 2 + lax.axis_index("c")

    def step(t, carry):
        chunk_id = wid + NSC * t

        @pl.when(chunk_id < NCHUNKS)
        def _():
            r0 = chunk_id * RPC
            pltpu.sync_copy(idx_hbm.at[pl.ds(r0, RPC), :], idx_buf)
            cps = [
                pltpu.async_copy(
                    ftab_hbm.at[idx_buf.at[r]],
                    rows_buf.at[pl.ds(r * IW, IW), :],
                    sem,
                )
                for r in range(RPC)
            ]
            for cp in cps:
                cp.wait()
            pltpu.sync_copy(
                rows_buf, gath_hbm.at[pl.ds(chunk_id * CHUNK, CHUNK), :]
            )

        return carry

    lax.fori_loop(0, (NCHUNKS + NSC - 1) // NSC, step, 0)


_JB = 8000


def _concat_body(rpe_ref, gath_ref, out_ref):
    out_ref[:, :, 0:C] = rpe_ref[...]
    out_ref[:, :, C : 2 * C] = gath_ref[...]


def _tc_concat(rpe_t, gath):
    return pl.pallas_call(
        _concat_body,
        grid=(B, NK // _JB),
        in_specs=[
            pl.BlockSpec((1, _JB, C), lambda b, j: (b, j, 0)),
            pl.BlockSpec((1, _JB, C), lambda b, j: (b, j, 0)),
        ],
        out_specs=pl.BlockSpec((1, _JB, 2 * C), lambda b, j: (b, j, 0)),
        out_shape=jax.ShapeDtypeStruct((B, NK, 2 * C), jnp.float32),
    )(rpe_t, gath)


def kernel(relative_position_encoding, features, neighbors):
    ftab = jnp.transpose(features[:, :, :, 0], (0, 2, 1)).reshape(B * N, C)
    rpe_t = jnp.transpose(relative_position_encoding, (0, 2, 3, 1)).reshape(
        B, NK, C
    )
    idxg = neighbors + (jnp.arange(B, dtype=jnp.int32) * N)[:, None, None]
    idxg = idxg.reshape(NROWS, IW)
    gath = _sc_gather(ftab, idxg)
    out = _tc_concat(rpe_t, gath.reshape(B, NK, C))
    return jnp.transpose(out.reshape(B, N, K, 2 * C), (0, 3, 1, 2))
